# baseline (speedup 1.0000x reference)
import functools

import jax
import jax.numpy as jnp
from jax import lax
from jax.experimental import pallas as pl
from jax.experimental.pallas import tpu as pltpu
from jax.experimental.pallas import tpu_sc as plsc

N = 10000
DIN = 256
HID = 128
BSEG = 512
LGEX = 651
E = 160000

NW = 32
CH = 64
NCHUNK = 80
EPAD = NW * NCHUNK * CH
NPAD = 10112
RPT = NPAD // 16
NROW = 10240

_HIGH = lax.Precision.HIGHEST


def _dotT(a, w):
    return lax.dot_general(a, w, (((1,), (1,)), ((), ())),
                           precision=_HIGH, preferred_element_type=jnp.float32)


def _make_segsum(width):
    mesh = plsc.VectorSubcoreMesh(core_axis_name="c", subcore_axis_name="s")

    @functools.partial(
        pl.kernel,
        out_type=jax.ShapeDtypeStruct((2, NPAD, width), jnp.float32),
        mesh=mesh,
        scratch_types=[
            pltpu.VMEM((NCHUNK, CH), jnp.int32),
            pltpu.VMEM((NCHUNK, CH), jnp.int32),
            pltpu.VMEM((3, CH, width), jnp.float32),
            pltpu.VMEM_SHARED((NPAD, width), jnp.float32),
            pltpu.SemaphoreType.DMA,
            pltpu.SemaphoreType.DMA,
        ],
    )
    def segsum(p_hbm, src_hbm, dst_hbm, zeros_hbm, out_hbm,
               src_v, dst_v, rows_v, acc_sh, gsem, ssem):
        c = lax.axis_index("c")
        s = lax.axis_index("s")
        wid = s * 2 + c
        rs = s * RPT
        pltpu.sync_copy(zeros_hbm.at[pl.ds(rs, RPT)], acc_sh.at[pl.ds(rs, RPT)])
        pltpu.sync_copy(src_hbm.at[wid], src_v)
        pltpu.sync_copy(dst_hbm.at[wid], dst_v)
        plsc.subcore_barrier()
        NBUF = 3
        handles = [None] * NBUF
        for j in range(NBUF - 1):
            handles[j] = pltpu.async_copy(
                p_hbm.at[src_v.at[j]], rows_v.at[j], gsem)
        for j in range(NCHUNK):
            nj = j + NBUF - 1
            if nj < NCHUNK:
                handles[nj % NBUF] = pltpu.async_copy(
                    p_hbm.at[src_v.at[nj]], rows_v.at[nj % NBUF], gsem)
            handles[j % NBUF].wait()
            pltpu.sync_copy(rows_v.at[j % NBUF], acc_sh.at[dst_v.at[j]],
                            add=True)
        plsc.subcore_barrier()
        pltpu.sync_copy(acc_sh.at[pl.ds(rs, RPT)],
                        out_hbm.at[c].at[pl.ds(rs, RPT)])

    return segsum


_segsum_cache = {}


def _edge_aggregate(table, srcr, dstr, zeros):
    width = table.shape[-1]
    if width not in _segsum_cache:
        _segsum_cache[width] = _make_segsum(width)
    return _segsum_cache[width](table, srcr, dstr, zeros)


def _proj1_body(x_ref, wl_ref, wr_ref, dr_ref, p_ref, q_ref, deg_ref):
    x = x_ref[...]
    p_ref[0:N, :] = _dotT(x, wl_ref[...])
    p_ref[N:NPAD, :] = jnp.zeros((NPAD - N, HID), jnp.float32)
    q_ref[...] = _dotT(x, wr_ref[...])
    _deg_body(dr_ref, deg_ref)


def _deg_body(dr_ref, out_ref):
    acc = jnp.zeros((128, 80), jnp.float32)
    CHE = 16000
    for j in range(E // CHE):
        sl = slice(j * CHE, (j + 1) * CHE)
        d = dr_ref[:, sl]
        lo = lax.broadcasted_iota(jnp.int32, (128, CHE), 0)
        hi = lax.broadcasted_iota(jnp.int32, (80, CHE), 0)
        a = (lo == (d & 127)).astype(jnp.bfloat16)
        b = ((d >> 7) == hi).astype(jnp.bfloat16)
        acc = acc + lax.dot_general(
            a, b, (((1,), (1,)), ((), ())),
            preferred_element_type=jnp.float32)
    out_ref[...] = acc


def _bn(h, g, be):
    mu = jnp.mean(h, axis=0, keepdims=True)
    var = jnp.mean((h - mu) ** 2, axis=0, keepdims=True)
    return g * (h - mu) * lax.rsqrt(var + 1e-5) + be


def _mid_body(agg_ref, q1_ref, deg_ref, b1_ref, g1_ref, be1_ref,
              w2l_ref, w2r_ref, p2_ref, q2_ref, degc_ref):
    feats = agg_ref[0, 0:N, :] + agg_ref[1, 0:N, :]
    degc = jnp.maximum(deg_ref[...], 1.0)
    h = feats / degc + b1_ref[...] + q1_ref[...]
    h = jnp.maximum(h, 0.0)
    hn = _bn(h, g1_ref[...], be1_ref[...])
    p2_ref[0:N, :] = _dotT(hn, w2l_ref[...])
    p2_ref[N:NPAD, :] = jnp.zeros((NPAD - N, HID), jnp.float32)
    q2_ref[...] = _dotT(hn, w2r_ref[...])
    degc_ref[...] = degc


def _final_body(agg_ref, q2_ref, degc_ref, b2_ref, g2_ref, be2_ref,
                idcol_ref, idrow_ref, out_ref):
    aggs = agg_ref[0, 0:N, :] + agg_ref[1, 0:N, :]
    h = aggs / degc_ref[...] + b2_ref[...] + q2_ref[...]
    h = jnp.maximum(h, 0.0)
    x = _bn(h, g2_ref[...], be2_ref[...])
    idc = idcol_ref[...]
    neg = jnp.float32(-jnp.inf)
    d = 1
    while d < N:
        xs = jnp.concatenate(
            [jnp.full((d, HID), neg, jnp.float32), x[0:N - d, :]], 0)
        ids_shift = jnp.concatenate(
            [jnp.full((d, 1), -1, jnp.int32), idc[0:N - d]], 0)
        x = jnp.maximum(x, jnp.where(ids_shift == idc, xs, neg))
        d *= 2
    xt = jnp.concatenate(
        [x, jnp.zeros((NROW - N, HID), jnp.float32)], 0)
    extra = jnp.concatenate(
        [jnp.ones((NROW, 1), jnp.float32), jnp.zeros((NROW, 15), jnp.float32)], 1)
    xt = jnp.concatenate([xt, extra], 1)
    idr = idrow_ref[...]
    nxt = jnp.concatenate(
        [idr[:, 1:], jnp.full((1, 1), jnp.int32(2 ** 30))], 1)
    is_end = idr != nxt
    acc = jnp.zeros((BSEG, 144), jnp.float32)
    CW = 2048
    for j in range(NROW // CW):
        sl = slice(j * CW, (j + 1) * CW)
        idchunk = idr[:, sl]
        iec = is_end[:, sl]
        iota_b = lax.broadcasted_iota(jnp.int32, (BSEG, CW), 0)
        m = jnp.where((iota_b == idchunk) & iec, 1.0, 0.0).astype(jnp.float32)
        acc = acc + lax.dot_general(
            m, xt[sl, :], (((1,), (0,)), ((), ())),
            precision=_HIGH, preferred_element_type=jnp.float32)
    cnt = acc[:, 128:129]
    out_ref[...] = jnp.where(cnt > 0.5, acc[:, 0:128], neg)


def _gexpr_body(g_ref, cw_ref, cb_ref, fw_ref, fb_ref, gc_ref, bec_ref,
                out_ref):
    g = g_ref[...]
    W636 = LGEX - 16 + 1
    chans = []
    for o in range(3):
        a = jnp.zeros((BSEG, W636), jnp.float32)
        for k in range(16):
            a = a + g[:, k:k + W636] * cw_ref[o, k]
        chans.append(a + cb_ref[o])
    xc = jnp.concatenate(chans, 1)
    y = _dotT(xc, fw_ref[...]) + fb_ref[...]
    out_ref[...] = _bn(y, gc_ref[...], bec_ref[...])


def _run_proj1(x, w1l, w1r, dst_row):
    return pl.pallas_call(
        _proj1_body,
        compiler_params=pltpu.CompilerParams(
            vmem_limit_bytes=64 * 1024 * 1024),
        out_shape=[jax.ShapeDtypeStruct((NPAD, HID), jnp.float32),
                   jax.ShapeDtypeStruct((N, HID), jnp.float32),
                   jax.ShapeDtypeStruct((128, 80), jnp.float32)],
    )(x, w1l, w1r, dst_row)


def _run_mid(agg, q1, deg, b1, g1, be1, w2l, w2r):
    return pl.pallas_call(
        _mid_body,
        compiler_params=pltpu.CompilerParams(
            vmem_limit_bytes=64 * 1024 * 1024),
        out_shape=[jax.ShapeDtypeStruct((NPAD, HID), jnp.float32),
                   jax.ShapeDtypeStruct((N, HID), jnp.float32),
                   jax.ShapeDtypeStruct((N, 1), jnp.float32)],
    )(agg, q1, deg, b1, g1, be1, w2l, w2r)


def _run_final(agg2, q2, degc, b2, g2, be2, idcol, idrow):
    return pl.pallas_call(
        _final_body,
        compiler_params=pltpu.CompilerParams(
            vmem_limit_bytes=64 * 1024 * 1024),
        out_shape=jax.ShapeDtypeStruct((BSEG, HID), jnp.float32),
    )(agg2, q2, degc, b2, g2, be2, idcol, idrow)


def _run_gexpr(gexpr, cw, cb, fw, fb, gc, bec):
    return pl.pallas_call(
        _gexpr_body,
        in_specs=[
            pl.BlockSpec(memory_space=pltpu.VMEM),
            pl.BlockSpec(memory_space=pltpu.SMEM),
            pl.BlockSpec(memory_space=pltpu.SMEM),
            pl.BlockSpec(memory_space=pltpu.VMEM),
            pl.BlockSpec(memory_space=pltpu.VMEM),
            pl.BlockSpec(memory_space=pltpu.VMEM),
            pl.BlockSpec(memory_space=pltpu.VMEM),
        ],
        out_shape=jax.ShapeDtypeStruct((BSEG, 100), jnp.float32),
    )(gexpr, cw, cb, fw, fb, gc, bec)


def kernel(drug_feature, drug_adj, ibatch, gexpr_data, W1l, b1l, W1r, g1, be1,
           W2l, b2l, W2r, g2, be2, conv_w, conv_b, fc_w, fc_b, gc, bec):
    src = drug_adj[0]
    dst = drug_adj[1]
    padi = jnp.full((EPAD - E,), N, jnp.int32)
    srcr = jnp.concatenate([src, padi]).reshape(NW, NCHUNK, CH)
    dstr = jnp.concatenate([dst, padi]).reshape(NW, NCHUNK, CH)
    zeros128 = jnp.zeros((NPAD, HID), jnp.float32)
    idcol = ibatch.reshape(N, 1)
    idrow = jnp.concatenate(
        [ibatch, jnp.full((NROW - N,), BSEG, jnp.int32)]).reshape(1, NROW)
    r1 = lambda v: v.reshape(1, -1)

    xc = _run_gexpr(gexpr_data, conv_w.reshape(3, 16), conv_b,
                    fc_w, r1(fc_b), r1(gc), r1(bec))

    p1, q1, deg2d = _run_proj1(drug_feature, W1l, W1r, dst.reshape(1, E))
    deg_col = deg2d.T.reshape(-1)[0:N].reshape(N, 1)
    agg1 = _edge_aggregate(p1, srcr, dstr, zeros128)
    p2, q2, degc = _run_mid(agg1, q1, deg_col, r1(b1l), r1(g1), r1(be1),
                            W2l, W2r)

    agg2 = _edge_aggregate(p2, srcr, dstr, zeros128)
    x_drug = _run_final(agg2, q2, degc, r1(b2l), r1(g2), r1(be2),
                        idcol, idrow)

    return (x_drug, xc)


# speedup vs baseline: 1.2569x; 1.2569x over previous
import functools

import jax
import jax.numpy as jnp
from jax import lax
from jax.experimental import pallas as pl
from jax.experimental.pallas import tpu as pltpu
from jax.experimental.pallas import tpu_sc as plsc

N = 10000
DIN = 256
HID = 128
BSEG = 512
LGEX = 651
E = 160000

NW = 32
CH = 64
NCHUNK = 80
EPAD = NW * NCHUNK * CH
NPAD = 10112
RPT = NPAD // 16
NROW = 10240

_HIGH = lax.Precision.HIGHEST


def _dotT(a, w):
    return lax.dot_general(a, w, (((1,), (1,)), ((), ())),
                           precision=_HIGH, preferred_element_type=jnp.float32)


def _make_segsum(width):
    mesh = plsc.VectorSubcoreMesh(core_axis_name="c", subcore_axis_name="s")

    @functools.partial(
        pl.kernel,
        out_type=jax.ShapeDtypeStruct((2, NPAD, width), jnp.float32),
        mesh=mesh,
        scratch_types=[
            pltpu.VMEM((NCHUNK, CH), jnp.int32),
            pltpu.VMEM((NCHUNK, CH), jnp.int32),
            pltpu.VMEM((3, CH, width), jnp.float32),
            pltpu.VMEM_SHARED((NPAD, width), jnp.float32),
            pltpu.SemaphoreType.DMA,
            pltpu.SemaphoreType.DMA,
        ],
    )
    def segsum(p_hbm, src_hbm, dst_hbm, zeros_hbm, out_hbm,
               src_v, dst_v, rows_v, acc_sh, gsem, ssem):
        c = lax.axis_index("c")
        s = lax.axis_index("s")
        wid = s * 2 + c
        rs = s * RPT
        pltpu.sync_copy(zeros_hbm.at[pl.ds(rs, RPT)], acc_sh.at[pl.ds(rs, RPT)])
        pltpu.sync_copy(src_hbm.at[wid], src_v)
        pltpu.sync_copy(dst_hbm.at[wid], dst_v)
        plsc.subcore_barrier()
        NBUF = 3
        handles = [None] * NBUF
        for j in range(NBUF - 1):
            handles[j] = pltpu.async_copy(
                p_hbm.at[src_v.at[j]], rows_v.at[j], gsem)
        for j in range(NCHUNK):
            nj = j + NBUF - 1
            if nj < NCHUNK:
                handles[nj % NBUF] = pltpu.async_copy(
                    p_hbm.at[src_v.at[nj]], rows_v.at[nj % NBUF], gsem)
            handles[j % NBUF].wait()
            pltpu.sync_copy(rows_v.at[j % NBUF], acc_sh.at[dst_v.at[j]],
                            add=True)
        plsc.subcore_barrier()
        pltpu.sync_copy(acc_sh.at[pl.ds(rs, RPT)],
                        out_hbm.at[c].at[pl.ds(rs, RPT)])

    return segsum


_segsum_cache = {}


def _edge_aggregate(table, srcr, dstr, zeros):
    width = table.shape[-1]
    if width not in _segsum_cache:
        _segsum_cache[width] = _make_segsum(width)
    return _segsum_cache[width](table, srcr, dstr, zeros)


def _proj1_body(x_ref, wl_ref, wr_ref, p_ref, q_ref):
    x = x_ref[...]
    p_ref[0:N, :] = _dotT(x, wl_ref[...])
    p_ref[N:NPAD, :] = jnp.zeros((NPAD - N, HID), jnp.float32)
    q_ref[...] = _dotT(x, wr_ref[...])


def _deg_body(dr_ref, out_ref):
    acc = jnp.zeros((128, 80), jnp.float32)
    CHE = 16000
    for j in range(E // CHE):
        sl = slice(j * CHE, (j + 1) * CHE)
        d = dr_ref[:, sl]
        lo = lax.broadcasted_iota(jnp.int32, (128, CHE), 0)
        hi = lax.broadcasted_iota(jnp.int32, (80, CHE), 0)
        a = (lo == (d & 127)).astype(jnp.bfloat16)
        b = ((d >> 7) == hi).astype(jnp.bfloat16)
        acc = acc + lax.dot_general(
            a, b, (((1,), (1,)), ((), ())),
            preferred_element_type=jnp.float32)
    out_ref[...] = acc


def _bn(h, g, be):
    mu = jnp.mean(h, axis=0, keepdims=True)
    var = jnp.mean((h - mu) ** 2, axis=0, keepdims=True)
    return g * (h - mu) * lax.rsqrt(var + 1e-5) + be


def _mid_body(agg_ref, q1_ref, deg_ref, b1_ref, g1_ref, be1_ref,
              w2l_ref, w2r_ref, p2_ref, q2_ref, degc_ref):
    feats = agg_ref[0, 0:N, :] + agg_ref[1, 0:N, :]
    degc = jnp.maximum(deg_ref[...], 1.0)
    h = feats / degc + b1_ref[...] + q1_ref[...]
    h = jnp.maximum(h, 0.0)
    hn = _bn(h, g1_ref[...], be1_ref[...])
    p2_ref[0:N, :] = _dotT(hn, w2l_ref[...])
    p2_ref[N:NPAD, :] = jnp.zeros((NPAD - N, HID), jnp.float32)
    q2_ref[...] = _dotT(hn, w2r_ref[...])
    degc_ref[...] = degc


def _final_body(agg_ref, q2_ref, degc_ref, b2_ref, g2_ref, be2_ref,
                idcol_ref, idrow_ref, out_ref):
    aggs = agg_ref[0, 0:N, :] + agg_ref[1, 0:N, :]
    h = aggs / degc_ref[...] + b2_ref[...] + q2_ref[...]
    h = jnp.maximum(h, 0.0)
    x = _bn(h, g2_ref[...], be2_ref[...])
    idc = idcol_ref[...]
    neg = jnp.float32(-jnp.inf)
    d = 1
    while d < N:
        xs = jnp.concatenate(
            [jnp.full((d, HID), neg, jnp.float32), x[0:N - d, :]], 0)
        ids_shift = jnp.concatenate(
            [jnp.full((d, 1), -1, jnp.int32), idc[0:N - d]], 0)
        x = jnp.maximum(x, jnp.where(ids_shift == idc, xs, neg))
        d *= 2
    xt = jnp.concatenate(
        [x, jnp.zeros((NROW - N, HID), jnp.float32)], 0)
    extra = jnp.concatenate(
        [jnp.ones((NROW, 1), jnp.float32), jnp.zeros((NROW, 15), jnp.float32)], 1)
    xt = jnp.concatenate([xt, extra], 1)
    idr = idrow_ref[...]
    nxt = jnp.concatenate(
        [idr[:, 1:], jnp.full((1, 1), jnp.int32(2 ** 30))], 1)
    is_end = idr != nxt
    acc = jnp.zeros((BSEG, 144), jnp.float32)
    CW = 2048
    for j in range(NROW // CW):
        sl = slice(j * CW, (j + 1) * CW)
        idchunk = idr[:, sl]
        iec = is_end[:, sl]
        iota_b = lax.broadcasted_iota(jnp.int32, (BSEG, CW), 0)
        m = jnp.where((iota_b == idchunk) & iec, 1.0, 0.0).astype(jnp.float32)
        acc = acc + lax.dot_general(
            m, xt[sl, :], (((1,), (0,)), ((), ())),
            precision=_HIGH, preferred_element_type=jnp.float32)
    cnt = acc[:, 128:129]
    out_ref[...] = jnp.where(cnt > 0.5, acc[:, 0:128], neg)


def _gexpr_body(g_ref, cw_ref, cb_ref, fw_ref, fb_ref, gc_ref, bec_ref,
                out_ref):
    g = g_ref[...]
    W636 = LGEX - 16 + 1
    chans = []
    for o in range(3):
        a = jnp.zeros((BSEG, W636), jnp.float32)
        for k in range(16):
            a = a + g[:, k:k + W636] * cw_ref[o, k]
        chans.append(a + cb_ref[o])
    xc = jnp.concatenate(chans, 1)
    y = _dotT(xc, fw_ref[...]) + fb_ref[...]
    out_ref[...] = _bn(y, gc_ref[...], bec_ref[...])


def _run_proj1(x, w1l, w1r):
    return pl.pallas_call(
        _proj1_body,
        compiler_params=pltpu.CompilerParams(
            vmem_limit_bytes=64 * 1024 * 1024),
        out_shape=[jax.ShapeDtypeStruct((NPAD, HID), jnp.float32),
                   jax.ShapeDtypeStruct((N, HID), jnp.float32)],
    )(x, w1l, w1r)


def _run_deg(dst_row):
    return pl.pallas_call(
        _deg_body,
        out_shape=jax.ShapeDtypeStruct((128, 80), jnp.float32),
    )(dst_row)


def _run_mid(agg, q1, deg, b1, g1, be1, w2l, w2r):
    return pl.pallas_call(
        _mid_body,
        compiler_params=pltpu.CompilerParams(
            vmem_limit_bytes=64 * 1024 * 1024),
        out_shape=[jax.ShapeDtypeStruct((NPAD, HID), jnp.float32),
                   jax.ShapeDtypeStruct((N, HID), jnp.float32),
                   jax.ShapeDtypeStruct((N, 1), jnp.float32)],
    )(agg, q1, deg, b1, g1, be1, w2l, w2r)


def _run_final(agg2, q2, degc, b2, g2, be2, idcol, idrow):
    return pl.pallas_call(
        _final_body,
        compiler_params=pltpu.CompilerParams(
            vmem_limit_bytes=64 * 1024 * 1024),
        out_shape=jax.ShapeDtypeStruct((BSEG, HID), jnp.float32),
    )(agg2, q2, degc, b2, g2, be2, idcol, idrow)


def _run_gexpr(gexpr, cw, cb, fw, fb, gc, bec):
    return pl.pallas_call(
        _gexpr_body,
        in_specs=[
            pl.BlockSpec(memory_space=pltpu.VMEM),
            pl.BlockSpec(memory_space=pltpu.SMEM),
            pl.BlockSpec(memory_space=pltpu.SMEM),
            pl.BlockSpec(memory_space=pltpu.VMEM),
            pl.BlockSpec(memory_space=pltpu.VMEM),
            pl.BlockSpec(memory_space=pltpu.VMEM),
            pl.BlockSpec(memory_space=pltpu.VMEM),
        ],
        out_shape=jax.ShapeDtypeStruct((BSEG, 100), jnp.float32),
    )(gexpr, cw, cb, fw, fb, gc, bec)


def kernel(drug_feature, drug_adj, ibatch, gexpr_data, W1l, b1l, W1r, g1, be1,
           W2l, b2l, W2r, g2, be2, conv_w, conv_b, fc_w, fc_b, gc, bec):
    src = drug_adj[0]
    dst = drug_adj[1]
    padi = jnp.full((EPAD - E,), N, jnp.int32)
    srcr = jnp.concatenate([src, padi]).reshape(NW, NCHUNK, CH)
    dstr = jnp.concatenate([dst, padi]).reshape(NW, NCHUNK, CH)
    zeros128 = jnp.zeros((NPAD, HID), jnp.float32)
    idcol = ibatch.reshape(N, 1)
    idrow = jnp.concatenate(
        [ibatch, jnp.full((NROW - N,), BSEG, jnp.int32)]).reshape(1, NROW)
    r1 = lambda v: v.reshape(1, -1)

    xc = _run_gexpr(gexpr_data, conv_w.reshape(3, 16), conv_b,
                    fc_w, r1(fc_b), r1(gc), r1(bec))

    deg2d = _run_deg(dst.reshape(1, E))
    deg_col = deg2d.T.reshape(-1)[0:N].reshape(N, 1)
    p1, q1 = _run_proj1(drug_feature, W1l, W1r)
    agg1 = _edge_aggregate(p1, srcr, dstr, zeros128)
    p2, q2, degc = _run_mid(agg1, q1, deg_col, r1(b1l), r1(g1), r1(be1),
                            W2l, W2r)

    agg2 = _edge_aggregate(p2, srcr, dstr, zeros128)
    x_drug = _run_final(agg2, q2, degc, r1(b2l), r1(g2), r1(be2),
                        idcol, idrow)

    return (x_drug, xc)
